# baseline (speedup 1.0000x reference)
import functools

import jax
import jax.numpy as jnp
from jax import lax
from jax.experimental import pallas as pl
from jax.experimental.pallas import tpu as pltpu
from jax.experimental.pallas import tpu_sc as plsc

_N = 4096
_E = 131072
_H = 4
_FIN = 256
_FOUT = 64
_D = _H * _FOUT

_NW = 32
_EPT = _E // _NW
_GRP = 16
_BN = 512

_SCP = pltpu.CompilerParams(needs_layout_passes=False)


def _proj_body(h_ref, w_ref, f_ref, hp_ref, aa_ref, md_ref, mscr):
    i = pl.program_id(0)
    hb = h_ref[...]
    hpb = jnp.dot(hb, w_ref[...], preferred_element_type=jnp.float32)
    hp_ref[...] = hpb.astype(jnp.bfloat16)
    aab = jnp.dot(hpb, f_ref[...], preferred_element_type=jnp.float32)
    aa_ref[...] = aab
    cm = jnp.max(aab, axis=0, keepdims=True)

    @pl.when(i == 0)
    def _():
        mscr[...] = cm

    @pl.when(i > 0)
    def _():
        mscr[...] = jnp.maximum(mscr[...], cm)

    md_ref[...] = mscr[...]


def _project(h, wcat, fcsd):
    nb = _N // _BN
    return pl.pallas_call(
        _proj_body,
        grid=(nb,),
        in_specs=[
            pl.BlockSpec((_BN, _FIN), lambda i: (i, 0)),
            pl.BlockSpec((_FIN, _D), lambda i: (0, 0)),
            pl.BlockSpec((_D, 16), lambda i: (0, 0)),
        ],
        out_specs=[
            pl.BlockSpec((_BN, _D), lambda i: (i, 0)),
            pl.BlockSpec((_BN, 16), lambda i: (i, 0)),
            pl.BlockSpec((1, 16), lambda i: (0, 0)),
        ],
        out_shape=[
            jax.ShapeDtypeStruct((_N, _D), jnp.bfloat16),
            jax.ShapeDtypeStruct((_N, 16), jnp.float32),
            jax.ShapeDtypeStruct((1, 16), jnp.float32),
        ],
        scratch_shapes=[pltpu.VMEM((1, 16), jnp.float32)],
    )(h, wcat, fcsd)


def _wgt_body(src_hbm, dst_hbm, aa_hbm, md_hbm, wgt_hbm, s_hbm,
              srcv, dstv, aav, mdv, spriv, wchunk):
    cid = lax.axis_index("c")
    sid = lax.axis_index("s")
    wid = cid * 16 + sid

    pltpu.sync_copy(src_hbm.at[pl.ds(wid * _EPT, _EPT)], srcv)
    pltpu.sync_copy(dst_hbm.at[pl.ds(wid * _EPT, _EPT)], dstv)
    pltpu.sync_copy(aa_hbm, aav)
    pltpu.sync_copy(md_hbm, mdv)

    z16 = jnp.zeros((16,), jnp.float32)

    def zs(i, _):
        spriv[pl.ds(i * 16, 16)] = z16
        return 0
    lax.fori_loop(0, _H * _N // 16, zs, 0)

    mdh = [plsc.load_gather(mdv, [jnp.full((16,), _H + h, jnp.int32)])
           for h in range(_H)]

    def body(g, _):
        off = g * _GRP
        si = srcv[pl.ds(off, _GRP)]
        di = dstv[pl.ds(off, _GRP)]
        si16 = si * 16
        di16 = di * 16
        for h in range(_H):
            a_s = plsc.load_gather(aav, [si16 + h])
            a_d = plsc.load_gather(aav, [di16 + (_H + h)])
            lgt = a_s + a_d
            lgt = jnp.maximum(lgt, 0.2 * lgt)
            ub = a_s + mdh[h]
            ub = jnp.maximum(ub, 0.2 * ub)
            wgt = jnp.exp(lgt - ub)
            wchunk[pl.ds(h * _EPT + off, 16)] = wgt
            plsc.addupdate_scatter(spriv, [si + (h * _N)], wgt)
        return 0

    lax.fori_loop(0, _EPT // _GRP, body, 0)

    for h in range(_H):
        pltpu.sync_copy(wchunk.at[pl.ds(h * _EPT, _EPT)],
                        wgt_hbm.at[h, pl.ds(wid * _EPT, _EPT)])
    pltpu.sync_copy(spriv, s_hbm.at[wid])


def _wgt_phase(src, dst, aa, md):
    mesh = plsc.VectorSubcoreMesh(core_axis_name="c", subcore_axis_name="s")
    kern = functools.partial(
        pl.kernel,
        out_type=[
            jax.ShapeDtypeStruct((_H, _E), jnp.float32),
            jax.ShapeDtypeStruct((_NW, _H * _N), jnp.float32),
        ],
        mesh=mesh,
        compiler_params=_SCP,
        scratch_types=[
            pltpu.VMEM((_EPT,), jnp.int32),
            pltpu.VMEM((_EPT,), jnp.int32),
            pltpu.VMEM((_N * 16,), jnp.float32),
            pltpu.VMEM((16,), jnp.float32),
            pltpu.VMEM((_H * _N,), jnp.float32),
            pltpu.VMEM((_H * _EPT,), jnp.float32),
        ],
    )(_wgt_body)
    return kern(src, dst, aa.reshape(-1), md.reshape(-1))


_NR = _N // _NW
_CHK = 2048
_NCK = _E // _CHK
_CAP = 6144
_DG = 32
_NB = 4


def _agg_body(src_hbm, dst_hbm, wgt_hbm, hp_hbm, sinv_hbm, un_hbm,
              srcv, dstv, wch, msrc, mdst, mw, rows, unacc, sinv_t, nwbuf,
              gsem, ssem):
    cid = lax.axis_index("c")
    sid = lax.axis_index("s")
    wid = cid * 16 + sid
    lo = wid * _NR

    for h in range(_H):
        pltpu.sync_copy(sinv_hbm.at[h, pl.ds(lo, _NR)],
                        sinv_t.at[pl.ds(h * _NR, _NR)])

    z16 = jnp.zeros((16,), jnp.float32)
    zi16 = jnp.zeros((16,), jnp.int32)

    def z_un(i, _):
        unacc[pl.ds(i * 16, 16)] = z16
        return 0
    lax.fori_loop(0, _NR * _FOUT // 16, z_un, 0)

    def z_m(i, _):
        msrc[pl.ds(i * 16, 16)] = zi16
        mdst[pl.ds(i * 16, 16)] = zi16
        return 0
    lax.fori_loop(0, _CAP // 16, z_m, 0)

    def z_w(i, _):
        mw[pl.ds(i * 16, 16)] = z16
        return 0
    lax.fori_loop(0, _H * _CAP // 16, z_w, 0)

    def stage_cps(ck, b):
        base = ck * _CHK
        return [
            pltpu.make_async_copy(src_hbm.at[pl.ds(base, _CHK)],
                                  srcv.at[b], ssem.at[b]),
            pltpu.make_async_copy(dst_hbm.at[pl.ds(base, _CHK)],
                                  dstv.at[b], ssem.at[b]),
        ] + [
            pltpu.make_async_copy(wgt_hbm.at[h, pl.ds(base, _CHK)],
                                  wch.at[b, pl.ds(h * _CHK, _CHK)], ssem.at[b])
            for h in range(_H)
        ]

    def stage(ck, b):
        for cp in stage_cps(ck, b):
            cp.start()

    stage(0, 0)

    def chunk_body(ck, ptr):
        b = lax.rem(ck, 2)
        for cp in stage_cps(ck, b):
            cp.wait()

        @pl.when(ck + 1 < _NCK)
        def _():
            stage(ck + 1, lax.rem(ck + 1, 2))

        def grp_body(g, p):
            for u in range(2):
                off = g * 2 * _GRP + u * _GRP
                si = srcv[b, pl.ds(off, _GRP)]
                m = (si >= lo) & (si < lo + _NR)
                cnt = plsc.all_reduce_population_count(m)[0]

                @pl.when(cnt > 0)
                def _(si=si, m=m, p=p, off=off):
                    di = dstv[b, pl.ds(off, _GRP)]
                    plsc.store_compressed(msrc.at[pl.ds(p, 16)],
                                          si - lo, mask=m)
                    plsc.store_compressed(mdst.at[pl.ds(p, 16)], di, mask=m)
                    for h in range(_H):
                        wv = wch[b, pl.ds(h * _CHK + off, _GRP)]
                        plsc.store_compressed(mw.at[pl.ds(h * _CAP + p, 16)],
                                              wv, mask=m)
                p = p + cnt
            return p

        return lax.fori_loop(0, _CHK // _GRP // 2, grp_body, ptr)

    nmatch = lax.fori_loop(0, _NCK, chunk_body, jnp.int32(0))

    ndg = (nmatch + (_DG - 1)) // _DG

    def issue(dg, b):
        pltpu.async_copy(hp_hbm.at[mdst.at[pl.ds(dg * _DG, _DG)]],
                         rows.at[b], gsem.at[b])

    for k in range(_NB - 1):
        @pl.when(k < ndg)
        def _(k=k):
            issue(k, k)

    def acc_body(dg, _):
        b = lax.rem(dg, _NB)
        p = dg * _DG
        pltpu.make_async_copy(hp_hbm.at[mdst.at[pl.ds(p, _DG)]],
                              rows.at[b], gsem.at[b]).wait()

        @pl.when(dg + (_NB - 1) < ndg)
        def _():
            issue(dg + (_NB - 1), lax.rem(dg + (_NB - 1), _NB))

        for s in range(_DG // _GRP):
            q = p + s * _GRP
            rl = msrc[pl.ds(q, _GRP)]
            rl64 = rl * _FOUT
            for e in range(_GRP):
                rloc = rl64[e]
                rle = rl[e]
                bws = [plsc.load_gather(
                    mw, [jnp.full((16,), h * _CAP + s * _GRP + e,
                                  jnp.int32) + p]) *
                    plsc.load_gather(
                    sinv_t, [jnp.full((16,), h * _NR, jnp.int32) + rle])
                    for h in range(_H)]
                for c2 in range(2):
                    accev = None
                    accod = None
                    for h in range(_H):
                        xi = rows[b, s * _GRP + e,
                                  pl.ds(h * 32 + c2 * 16, 16)]
                        ev = plsc.bitcast(xi << 16, jnp.float32) * bws[h]
                        od = plsc.bitcast(xi & jnp.int32(-65536),
                                          jnp.float32) * bws[h]
                        accev = ev if accev is None else accev + ev
                        accod = od if accod is None else accod + od
                    base = rloc + c2 * 32
                    plsc.addupdate(unacc.at[pl.ds(base, 16)], accev)
                    plsc.addupdate(unacc.at[pl.ds(base + 16, 16)], accod)
        return 0

    lax.fori_loop(0, ndg, acc_body, 0)

    pltpu.sync_copy(unacc, un_hbm.at[pl.ds(wid * _NR * _FOUT, _NR * _FOUT)])


def _agg_phase(src, dst, wgt, hp, sinv):
    mesh = plsc.VectorSubcoreMesh(core_axis_name="c", subcore_axis_name="s")
    kern = functools.partial(
        pl.kernel,
        out_type=jax.ShapeDtypeStruct((_N * _FOUT,), jnp.float32),
        mesh=mesh,
        compiler_params=_SCP,
        scratch_types=[
            pltpu.VMEM((2, _CHK), jnp.int32),
            pltpu.VMEM((2, _CHK), jnp.int32),
            pltpu.VMEM((2, _H * _CHK), jnp.float32),
            pltpu.VMEM((_CAP,), jnp.int32),
            pltpu.VMEM((_CAP,), jnp.int32),
            pltpu.VMEM((_H * _CAP,), jnp.float32),
            pltpu.VMEM((_NB, _DG, _D // 2), jnp.int32),
            pltpu.VMEM((_NR * _FOUT,), jnp.float32),
            pltpu.VMEM((_H * _NR,), jnp.float32),
            pltpu.VMEM((_H * 16,), jnp.float32),
            pltpu.SemaphoreType.DMA((_NB,)),
            pltpu.SemaphoreType.DMA((2,)),
        ],
    )(_agg_body)
    return kern(src, dst, wgt, hp, sinv)


def _sinv_body(s_ref, o_ref):
    st = jnp.sum(s_ref[...], axis=0)
    o_ref[...] = jnp.where(st > 0.0, (1.0 / _H) / st, 0.0)


def _sinv(s_all):
    return pl.pallas_call(
        _sinv_body,
        in_specs=[pl.BlockSpec((_NW, _H, _N), lambda: (0, 0, 0))],
        out_specs=pl.BlockSpec((_H, _N), lambda: (0, 0)),
        out_shape=jax.ShapeDtypeStruct((_H, _N), jnp.float32),
    )(s_all)


def _fin_body(un_ref, b_ref, o_ref):
    un = un_ref[...]
    un = un.reshape(_BN, 2, 2, 16).transpose(0, 1, 3, 2)
    un = un.reshape(_BN, _FOUT)
    o_ref[...] = un + b_ref[...][None, :]


def _finalize(un, bias):
    nb = _N // _BN
    return pl.pallas_call(
        _fin_body,
        grid=(nb,),
        in_specs=[
            pl.BlockSpec((_BN, _FOUT), lambda i: (i, 0)),
            pl.BlockSpec((_FOUT,), lambda i: (0,)),
        ],
        out_specs=pl.BlockSpec((_BN, _FOUT), lambda i: (i, 0)),
        out_shape=jax.ShapeDtypeStruct((_N, _FOUT), jnp.float32),
    )(un, bias)


def kernel(h, edge_index, w, fc, bias):
    src = edge_index[0]
    dst = edge_index[1]
    wcat = jnp.transpose(w, (1, 0, 2)).reshape(_FIN, _D)
    fc_src = fc[:, :_FOUT, 0]
    fc_dst = fc[:, _FOUT:, 0]
    fcsd = jnp.zeros((_D, 16), jnp.float32)
    for hh in range(_H):
        fcsd = fcsd.at[hh * _FOUT:(hh + 1) * _FOUT, hh].set(fc_src[hh])
        fcsd = fcsd.at[hh * _FOUT:(hh + 1) * _FOUT, _H + hh].set(fc_dst[hh])

    hp16, aa, md = _project(h, wcat, fcsd)
    hp32 = lax.bitcast_convert_type(hp16.reshape(_N, _D // 2, 2), jnp.int32)
    wgt, s_all = _wgt_phase(src, dst, aa, md)
    sinv = _sinv(s_all.reshape(_NW, _H, _N))
    un = _agg_phase(src, dst, wgt, hp32, sinv)
    return _finalize(un.reshape(_N, _FOUT), bias)


# speedup vs baseline: 1.1465x; 1.1465x over previous
import functools

import jax
import jax.numpy as jnp
from jax import lax
from jax.experimental import pallas as pl
from jax.experimental.pallas import tpu as pltpu
from jax.experimental.pallas import tpu_sc as plsc

_N = 4096
_E = 131072
_H = 4
_FIN = 256
_FOUT = 64
_D = _H * _FOUT

_NW = 32
_EPT = _E // _NW
_GRP = 16
_BN = 512

_SCP = pltpu.CompilerParams(needs_layout_passes=False)


def _proj_body(h_ref, w_ref, f_ref, hp_ref, aa_ref, md_ref, mscr):
    i = pl.program_id(0)
    hb = h_ref[...]
    hpb = jnp.dot(hb, w_ref[...], preferred_element_type=jnp.float32)
    hp_ref[...] = hpb.astype(jnp.bfloat16)
    aab = jnp.dot(hpb, f_ref[...], preferred_element_type=jnp.float32)
    aa_ref[...] = aab
    cm = jnp.max(aab, axis=0, keepdims=True)

    @pl.when(i == 0)
    def _():
        mscr[...] = cm

    @pl.when(i > 0)
    def _():
        mscr[...] = jnp.maximum(mscr[...], cm)

    md_ref[...] = mscr[...]


def _project(h, wcat, fcsd):
    nb = _N // _BN
    return pl.pallas_call(
        _proj_body,
        grid=(nb,),
        in_specs=[
            pl.BlockSpec((_BN, _FIN), lambda i: (i, 0)),
            pl.BlockSpec((_FIN, _D), lambda i: (0, 0)),
            pl.BlockSpec((_D, 16), lambda i: (0, 0)),
        ],
        out_specs=[
            pl.BlockSpec((_BN, _D), lambda i: (i, 0)),
            pl.BlockSpec((_BN, 16), lambda i: (i, 0)),
            pl.BlockSpec((1, 16), lambda i: (0, 0)),
        ],
        out_shape=[
            jax.ShapeDtypeStruct((_N, _D), jnp.bfloat16),
            jax.ShapeDtypeStruct((_N, 16), jnp.float32),
            jax.ShapeDtypeStruct((1, 16), jnp.float32),
        ],
        scratch_shapes=[pltpu.VMEM((1, 16), jnp.float32)],
    )(h, wcat, fcsd)


def _wgt_body(src_hbm, dst_hbm, aa_hbm, md_hbm, wgt_hbm, s_hbm,
              srcv, dstv, aav, mdv, spriv, wchunk):
    cid = lax.axis_index("c")
    sid = lax.axis_index("s")
    wid = cid * 16 + sid

    pltpu.sync_copy(src_hbm.at[pl.ds(wid * _EPT, _EPT)], srcv)
    pltpu.sync_copy(dst_hbm.at[pl.ds(wid * _EPT, _EPT)], dstv)
    pltpu.sync_copy(aa_hbm, aav)
    pltpu.sync_copy(md_hbm, mdv)

    z16 = jnp.zeros((16,), jnp.float32)

    def zs(i, _):
        spriv[pl.ds(i * 16, 16)] = z16
        return 0
    lax.fori_loop(0, _H * _N // 16, zs, 0)

    mdh = [plsc.load_gather(mdv, [jnp.full((16,), _H + h, jnp.int32)])
           for h in range(_H)]

    def body(g, _):
        off = g * _GRP
        si = srcv[pl.ds(off, _GRP)]
        di = dstv[pl.ds(off, _GRP)]
        si16 = si * 16
        di16 = di * 16
        for h in range(_H):
            a_s = plsc.load_gather(aav, [si16 + h])
            a_d = plsc.load_gather(aav, [di16 + (_H + h)])
            lgt = a_s + a_d
            lgt = jnp.maximum(lgt, 0.2 * lgt)
            ub = a_s + mdh[h]
            ub = jnp.maximum(ub, 0.2 * ub)
            wgt = jnp.exp(lgt - ub)
            wchunk[pl.ds(h * _EPT + off, 16)] = wgt
            plsc.addupdate_scatter(spriv, [si + (h * _N)], wgt)
        return 0

    lax.fori_loop(0, _EPT // _GRP, body, 0)

    for h in range(_H):
        pltpu.sync_copy(wchunk.at[pl.ds(h * _EPT, _EPT)],
                        wgt_hbm.at[h, pl.ds(wid * _EPT, _EPT)])
    pltpu.sync_copy(spriv, s_hbm.at[wid])


def _wgt_phase(src, dst, aa, md):
    mesh = plsc.VectorSubcoreMesh(core_axis_name="c", subcore_axis_name="s")
    kern = functools.partial(
        pl.kernel,
        out_type=[
            jax.ShapeDtypeStruct((_H, _E), jnp.float32),
            jax.ShapeDtypeStruct((_NW, _H * _N), jnp.float32),
        ],
        mesh=mesh,
        compiler_params=_SCP,
        scratch_types=[
            pltpu.VMEM((_EPT,), jnp.int32),
            pltpu.VMEM((_EPT,), jnp.int32),
            pltpu.VMEM((_N * 16,), jnp.float32),
            pltpu.VMEM((16,), jnp.float32),
            pltpu.VMEM((_H * _N,), jnp.float32),
            pltpu.VMEM((_H * _EPT,), jnp.float32),
        ],
    )(_wgt_body)
    return kern(src, dst, aa.reshape(-1), md.reshape(-1))


_NR = _N // _NW
_CHK = 2048
_NCK = _E // _CHK
_CAP = 6144
_DG = 32
_NB = 4


def _agg_body(src_hbm, dst_hbm, wgt_hbm, hp_hbm, sinv_hbm, un_hbm,
              srcv, dstv, wch, msrc, mdst, mw, rows, unacc, sinv_t, nwbuf,
              gsem, ssem):
    cid = lax.axis_index("c")
    sid = lax.axis_index("s")
    wid = cid * 16 + sid
    lo = wid * _NR

    for h in range(_H):
        pltpu.sync_copy(sinv_hbm.at[h, pl.ds(lo, _NR)],
                        sinv_t.at[pl.ds(h * _NR, _NR)])

    z16 = jnp.zeros((16,), jnp.float32)
    zi16 = jnp.zeros((16,), jnp.int32)

    def z_un(i, _):
        unacc[pl.ds(i * 16, 16)] = z16
        return 0
    lax.fori_loop(0, _NR * _FOUT // 16, z_un, 0)

    def z_m(i, _):
        msrc[pl.ds(i * 16, 16)] = zi16
        mdst[pl.ds(i * 16, 16)] = zi16
        return 0
    lax.fori_loop(0, _CAP // 16, z_m, 0)

    def z_w(i, _):
        mw[pl.ds(i * 16, 16)] = z16
        return 0
    lax.fori_loop(0, _H * _CAP // 16, z_w, 0)

    def stage_cps(ck, b):
        base = ck * _CHK
        return [
            pltpu.make_async_copy(src_hbm.at[pl.ds(base, _CHK)],
                                  srcv.at[b], ssem.at[b]),
            pltpu.make_async_copy(dst_hbm.at[pl.ds(base, _CHK)],
                                  dstv.at[b], ssem.at[b]),
        ] + [
            pltpu.make_async_copy(wgt_hbm.at[h, pl.ds(base, _CHK)],
                                  wch.at[b, pl.ds(h * _CHK, _CHK)], ssem.at[b])
            for h in range(_H)
        ]

    def stage(ck, b):
        for cp in stage_cps(ck, b):
            cp.start()

    stage(0, 0)

    def chunk_body(ck, ptr):
        b = lax.rem(ck, 2)
        for cp in stage_cps(ck, b):
            cp.wait()

        @pl.when(ck + 1 < _NCK)
        def _():
            stage(ck + 1, lax.rem(ck + 1, 2))

        def grp_body(g, p):
            off = g * _GRP
            si = srcv[b, pl.ds(off, _GRP)]
            m = (si >= lo) & (si < lo + _NR)
            cnt = plsc.all_reduce_population_count(m)[0]
            di = dstv[b, pl.ds(off, _GRP)]
            plsc.store_compressed(msrc.at[pl.ds(p, 16)], si - lo, mask=m)
            plsc.store_compressed(mdst.at[pl.ds(p, 16)], di, mask=m)
            for h in range(_H):
                wv = wch[b, pl.ds(h * _CHK + off, _GRP)]
                plsc.store_compressed(mw.at[pl.ds(h * _CAP + p, 16)],
                                      wv, mask=m)
            return p + cnt

        return lax.fori_loop(0, _CHK // _GRP, grp_body, ptr)

    nmatch = lax.fori_loop(0, _NCK, chunk_body, jnp.int32(0))

    ndg = (nmatch + (_DG - 1)) // _DG

    def issue(dg, b):
        pltpu.async_copy(hp_hbm.at[mdst.at[pl.ds(dg * _DG, _DG)]],
                         rows.at[b], gsem.at[b])

    for k in range(_NB - 1):
        @pl.when(k < ndg)
        def _(k=k):
            issue(k, k)

    def acc_body(dg, _):
        b = lax.rem(dg, _NB)
        p = dg * _DG
        pltpu.make_async_copy(hp_hbm.at[mdst.at[pl.ds(p, _DG)]],
                              rows.at[b], gsem.at[b]).wait()

        @pl.when(dg + (_NB - 1) < ndg)
        def _():
            issue(dg + (_NB - 1), lax.rem(dg + (_NB - 1), _NB))

        for s in range(_DG // _GRP):
            q = p + s * _GRP
            rl = msrc[pl.ds(q, _GRP)]
            rl64 = rl * _FOUT
            for e in range(_GRP):
                rloc = rl64[e]
                rle = rl[e]
                bws = [plsc.load_gather(
                    mw, [jnp.full((16,), h * _CAP + s * _GRP + e,
                                  jnp.int32) + p]) *
                    plsc.load_gather(
                    sinv_t, [jnp.full((16,), h * _NR, jnp.int32) + rle])
                    for h in range(_H)]
                for c2 in range(2):
                    accev = None
                    accod = None
                    for h in range(_H):
                        xi = rows[b, s * _GRP + e,
                                  pl.ds(h * 32 + c2 * 16, 16)]
                        ev = plsc.bitcast(xi << 16, jnp.float32) * bws[h]
                        od = plsc.bitcast(xi & jnp.int32(-65536),
                                          jnp.float32) * bws[h]
                        accev = ev if accev is None else accev + ev
                        accod = od if accod is None else accod + od
                    base = rloc + c2 * 32
                    plsc.addupdate(unacc.at[pl.ds(base, 16)], accev)
                    plsc.addupdate(unacc.at[pl.ds(base + 16, 16)], accod)
        return 0

    lax.fori_loop(0, ndg, acc_body, 0)

    pltpu.sync_copy(unacc, un_hbm.at[pl.ds(wid * _NR * _FOUT, _NR * _FOUT)])


def _agg_phase(src, dst, wgt, hp, sinv):
    mesh = plsc.VectorSubcoreMesh(core_axis_name="c", subcore_axis_name="s")
    kern = functools.partial(
        pl.kernel,
        out_type=jax.ShapeDtypeStruct((_N * _FOUT,), jnp.float32),
        mesh=mesh,
        compiler_params=_SCP,
        scratch_types=[
            pltpu.VMEM((2, _CHK), jnp.int32),
            pltpu.VMEM((2, _CHK), jnp.int32),
            pltpu.VMEM((2, _H * _CHK), jnp.float32),
            pltpu.VMEM((_CAP,), jnp.int32),
            pltpu.VMEM((_CAP,), jnp.int32),
            pltpu.VMEM((_H * _CAP,), jnp.float32),
            pltpu.VMEM((_NB, _DG, _D // 2), jnp.int32),
            pltpu.VMEM((_NR * _FOUT,), jnp.float32),
            pltpu.VMEM((_H * _NR,), jnp.float32),
            pltpu.VMEM((_H * 16,), jnp.float32),
            pltpu.SemaphoreType.DMA((_NB,)),
            pltpu.SemaphoreType.DMA((2,)),
        ],
    )(_agg_body)
    return kern(src, dst, wgt, hp, sinv)


def _sinv_body(s_ref, o_ref):
    st = jnp.sum(s_ref[...], axis=0)
    o_ref[...] = jnp.where(st > 0.0, (1.0 / _H) / st, 0.0)


def _sinv(s_all):
    return pl.pallas_call(
        _sinv_body,
        in_specs=[pl.BlockSpec((_NW, _H, _N), lambda: (0, 0, 0))],
        out_specs=pl.BlockSpec((_H, _N), lambda: (0, 0)),
        out_shape=jax.ShapeDtypeStruct((_H, _N), jnp.float32),
    )(s_all)


def _fin_body(un_ref, b_ref, o_ref):
    un = un_ref[...]
    un = un.reshape(_BN, 2, 2, 16).transpose(0, 1, 3, 2)
    un = un.reshape(_BN, _FOUT)
    o_ref[...] = un + b_ref[...][None, :]


def _finalize(un, bias):
    nb = _N // _BN
    return pl.pallas_call(
        _fin_body,
        grid=(nb,),
        in_specs=[
            pl.BlockSpec((_BN, _FOUT), lambda i: (i, 0)),
            pl.BlockSpec((_FOUT,), lambda i: (0,)),
        ],
        out_specs=pl.BlockSpec((_BN, _FOUT), lambda i: (i, 0)),
        out_shape=jax.ShapeDtypeStruct((_N, _FOUT), jnp.float32),
    )(un, bias)


def kernel(h, edge_index, w, fc, bias):
    src = edge_index[0]
    dst = edge_index[1]
    wcat = jnp.transpose(w, (1, 0, 2)).reshape(_FIN, _D)
    fc_src = fc[:, :_FOUT, 0]
    fc_dst = fc[:, _FOUT:, 0]
    fcsd = jnp.zeros((_D, 16), jnp.float32)
    for hh in range(_H):
        fcsd = fcsd.at[hh * _FOUT:(hh + 1) * _FOUT, hh].set(fc_src[hh])
        fcsd = fcsd.at[hh * _FOUT:(hh + 1) * _FOUT, _H + hh].set(fc_dst[hh])

    hp16, aa, md = _project(h, wcat, fcsd)
    hp32 = lax.bitcast_convert_type(hp16.reshape(_N, _D // 2, 2), jnp.int32)
    wgt, s_all = _wgt_phase(src, dst, aa, md)
    sinv = _sinv(s_all.reshape(_NW, _H, _N))
    un = _agg_phase(src, dst, wgt, hp32, sinv)
    return _finalize(un.reshape(_N, _FOUT), bias)


# speedup vs baseline: 1.1516x; 1.0044x over previous
import functools

import jax
import jax.numpy as jnp
from jax import lax
from jax.experimental import pallas as pl
from jax.experimental.pallas import tpu as pltpu
from jax.experimental.pallas import tpu_sc as plsc

_N = 4096
_E = 131072
_H = 4
_FIN = 256
_FOUT = 64
_D = _H * _FOUT

_NW = 32
_EPT = _E // _NW
_GRP = 16
_BN = 512

_SCP = pltpu.CompilerParams(needs_layout_passes=False)


def _proj_body(h_ref, w_ref, f_ref, hp_ref, aa_ref, md_ref, mscr):
    i = pl.program_id(0)
    hb = h_ref[...]
    hpb = jnp.dot(hb, w_ref[...], preferred_element_type=jnp.float32)
    hp_ref[...] = hpb.astype(jnp.bfloat16)
    aab = jnp.dot(hpb, f_ref[...], preferred_element_type=jnp.float32)
    aa_ref[...] = aab
    cm = jnp.max(aab, axis=0, keepdims=True)

    @pl.when(i == 0)
    def _():
        mscr[...] = cm

    @pl.when(i > 0)
    def _():
        mscr[...] = jnp.maximum(mscr[...], cm)

    md_ref[...] = mscr[...]


def _project(h, wcat, fcsd):
    nb = _N // _BN
    return pl.pallas_call(
        _proj_body,
        grid=(nb,),
        in_specs=[
            pl.BlockSpec((_BN, _FIN), lambda i: (i, 0)),
            pl.BlockSpec((_FIN, _D), lambda i: (0, 0)),
            pl.BlockSpec((_D, 16), lambda i: (0, 0)),
        ],
        out_specs=[
            pl.BlockSpec((_BN, _D), lambda i: (i, 0)),
            pl.BlockSpec((_BN, 16), lambda i: (i, 0)),
            pl.BlockSpec((1, 16), lambda i: (0, 0)),
        ],
        out_shape=[
            jax.ShapeDtypeStruct((_N, _D), jnp.bfloat16),
            jax.ShapeDtypeStruct((_N, 16), jnp.float32),
            jax.ShapeDtypeStruct((1, 16), jnp.float32),
        ],
        scratch_shapes=[pltpu.VMEM((1, 16), jnp.float32)],
    )(h, wcat, fcsd)


def _wgt_body(src_hbm, dst_hbm, aa_hbm, md_hbm, wgt_hbm, s_hbm,
              srcv, dstv, aav, mdv, spriv, wchunk):
    cid = lax.axis_index("c")
    sid = lax.axis_index("s")
    wid = cid * 16 + sid

    pltpu.sync_copy(src_hbm.at[pl.ds(wid * _EPT, _EPT)], srcv)
    pltpu.sync_copy(dst_hbm.at[pl.ds(wid * _EPT, _EPT)], dstv)
    pltpu.sync_copy(aa_hbm, aav)
    pltpu.sync_copy(md_hbm, mdv)

    z16 = jnp.zeros((16,), jnp.float32)

    def zs(i, _):
        spriv[pl.ds(i * 16, 16)] = z16
        return 0
    lax.fori_loop(0, _H * _N // 16, zs, 0)

    mdh = [plsc.load_gather(mdv, [jnp.full((16,), _H + h, jnp.int32)])
           for h in range(_H)]

    def body(g, _):
        off = g * _GRP
        si = srcv[pl.ds(off, _GRP)]
        di = dstv[pl.ds(off, _GRP)]
        si16 = si * 16
        di16 = di * 16
        for h in range(_H):
            a_s = plsc.load_gather(aav, [si16 + h])
            a_d = plsc.load_gather(aav, [di16 + (_H + h)])
            lgt = a_s + a_d
            lgt = jnp.maximum(lgt, 0.2 * lgt)
            ub = a_s + mdh[h]
            ub = jnp.maximum(ub, 0.2 * ub)
            wgt = jnp.exp(lgt - ub)
            wchunk[pl.ds(h * _EPT + off, 16)] = wgt
            plsc.addupdate_scatter(spriv, [si + (h * _N)], wgt)
        return 0

    lax.fori_loop(0, _EPT // _GRP, body, 0)

    for h in range(_H):
        pltpu.sync_copy(wchunk.at[pl.ds(h * _EPT, _EPT)],
                        wgt_hbm.at[h, pl.ds(wid * _EPT, _EPT)])
    pltpu.sync_copy(spriv, s_hbm.at[wid])


def _wgt_phase(src, dst, aa, md):
    mesh = plsc.VectorSubcoreMesh(core_axis_name="c", subcore_axis_name="s")
    kern = functools.partial(
        pl.kernel,
        out_type=[
            jax.ShapeDtypeStruct((_H, _E), jnp.float32),
            jax.ShapeDtypeStruct((_NW, _H * _N), jnp.float32),
        ],
        mesh=mesh,
        compiler_params=_SCP,
        scratch_types=[
            pltpu.VMEM((_EPT,), jnp.int32),
            pltpu.VMEM((_EPT,), jnp.int32),
            pltpu.VMEM((_N * 16,), jnp.float32),
            pltpu.VMEM((16,), jnp.float32),
            pltpu.VMEM((_H * _N,), jnp.float32),
            pltpu.VMEM((_H * _EPT,), jnp.float32),
        ],
    )(_wgt_body)
    return kern(src, dst, aa.reshape(-1), md.reshape(-1))


_NR = _N // _NW
_CHK = 2048
_NCK = _E // _CHK
_CAP = 6144
_DG = 32
_NB = 4


def _agg_body(src_hbm, dst_hbm, wgt_hbm, hp_hbm, sinv_hbm, un_hbm,
              srcv, dstv, wch, msrc, mdst, mw, rows, unacc, sinv_t, nwbuf,
              gsem, ssem):
    cid = lax.axis_index("c")
    sid = lax.axis_index("s")
    wid = cid * 16 + sid
    lo = wid * _NR

    for h in range(_H):
        pltpu.sync_copy(sinv_hbm.at[h, pl.ds(lo, _NR)],
                        sinv_t.at[pl.ds(h * _NR, _NR)])

    z16 = jnp.zeros((16,), jnp.float32)
    zi16 = jnp.zeros((16,), jnp.int32)

    def z_un(i, _):
        unacc[pl.ds(i * 16, 16)] = z16
        return 0
    lax.fori_loop(0, _NR * _FOUT // 16, z_un, 0)

    def z_m(i, _):
        msrc[pl.ds(i * 16, 16)] = zi16
        mdst[pl.ds(i * 16, 16)] = zi16
        return 0
    lax.fori_loop(0, _CAP // 16, z_m, 0)

    def z_w(i, _):
        mw[pl.ds(i * 16, 16)] = z16
        return 0
    lax.fori_loop(0, _H * _CAP // 16, z_w, 0)

    def stage_cps(ck, b):
        base = ck * _CHK
        return [
            pltpu.make_async_copy(src_hbm.at[pl.ds(base, _CHK)],
                                  srcv.at[b], ssem.at[b]),
            pltpu.make_async_copy(dst_hbm.at[pl.ds(base, _CHK)],
                                  dstv.at[b], ssem.at[b]),
        ] + [
            pltpu.make_async_copy(wgt_hbm.at[h, pl.ds(base, _CHK)],
                                  wch.at[b, pl.ds(h * _CHK, _CHK)], ssem.at[b])
            for h in range(_H)
        ]

    def stage(ck, b):
        for cp in stage_cps(ck, b):
            cp.start()

    stage(0, 0)

    def chunk_body(ck, ptr):
        b = lax.rem(ck, 2)
        for cp in stage_cps(ck, b):
            cp.wait()

        @pl.when(ck + 1 < _NCK)
        def _():
            stage(ck + 1, lax.rem(ck + 1, 2))

        def grp_body(g, p):
            for u in range(2):
                off = g * 2 * _GRP + u * _GRP
                si = srcv[b, pl.ds(off, _GRP)]
                m = (si >= lo) & (si < lo + _NR)
                cnt = plsc.all_reduce_population_count(m)[0]
                di = dstv[b, pl.ds(off, _GRP)]
                plsc.store_compressed(msrc.at[pl.ds(p, 16)], si - lo, mask=m)
                plsc.store_compressed(mdst.at[pl.ds(p, 16)], di, mask=m)
                for h in range(_H):
                    wv = wch[b, pl.ds(h * _CHK + off, _GRP)]
                    plsc.store_compressed(mw.at[pl.ds(h * _CAP + p, 16)],
                                          wv, mask=m)
                p = p + cnt
            return p

        return lax.fori_loop(0, _CHK // _GRP // 2, grp_body, ptr)

    nmatch = lax.fori_loop(0, _NCK, chunk_body, jnp.int32(0))

    ndg = (nmatch + (_DG - 1)) // _DG

    def issue(dg, b):
        pltpu.async_copy(hp_hbm.at[mdst.at[pl.ds(dg * _DG, _DG)]],
                         rows.at[b], gsem.at[b])

    for k in range(_NB - 1):
        @pl.when(k < ndg)
        def _(k=k):
            issue(k, k)

    def acc_body(dg, _):
        b = lax.rem(dg, _NB)
        p = dg * _DG
        pltpu.make_async_copy(hp_hbm.at[mdst.at[pl.ds(p, _DG)]],
                              rows.at[b], gsem.at[b]).wait()

        @pl.when(dg + (_NB - 1) < ndg)
        def _():
            issue(dg + (_NB - 1), lax.rem(dg + (_NB - 1), _NB))

        for s in range(_DG // _GRP):
            q = p + s * _GRP
            rl = msrc[pl.ds(q, _GRP)]
            rl64 = rl * _FOUT
            for e in range(_GRP):
                rloc = rl64[e]
                rle = rl[e]
                bws = [plsc.load_gather(
                    mw, [jnp.full((16,), h * _CAP + s * _GRP + e,
                                  jnp.int32) + p]) *
                    plsc.load_gather(
                    sinv_t, [jnp.full((16,), h * _NR, jnp.int32) + rle])
                    for h in range(_H)]
                for c2 in range(2):
                    accev = None
                    accod = None
                    for h in range(_H):
                        xi = rows[b, s * _GRP + e,
                                  pl.ds(h * 32 + c2 * 16, 16)]
                        ev = plsc.bitcast(xi << 16, jnp.float32) * bws[h]
                        od = plsc.bitcast(xi & jnp.int32(-65536),
                                          jnp.float32) * bws[h]
                        accev = ev if accev is None else accev + ev
                        accod = od if accod is None else accod + od
                    base = rloc + c2 * 32
                    plsc.addupdate(unacc.at[pl.ds(base, 16)], accev)
                    plsc.addupdate(unacc.at[pl.ds(base + 16, 16)], accod)
        return 0

    lax.fori_loop(0, ndg, acc_body, 0)

    pltpu.sync_copy(unacc, un_hbm.at[pl.ds(wid * _NR * _FOUT, _NR * _FOUT)])


def _agg_phase(src, dst, wgt, hp, sinv):
    mesh = plsc.VectorSubcoreMesh(core_axis_name="c", subcore_axis_name="s")
    kern = functools.partial(
        pl.kernel,
        out_type=jax.ShapeDtypeStruct((_N * _FOUT,), jnp.float32),
        mesh=mesh,
        compiler_params=_SCP,
        scratch_types=[
            pltpu.VMEM((2, _CHK), jnp.int32),
            pltpu.VMEM((2, _CHK), jnp.int32),
            pltpu.VMEM((2, _H * _CHK), jnp.float32),
            pltpu.VMEM((_CAP,), jnp.int32),
            pltpu.VMEM((_CAP,), jnp.int32),
            pltpu.VMEM((_H * _CAP,), jnp.float32),
            pltpu.VMEM((_NB, _DG, _D // 2), jnp.int32),
            pltpu.VMEM((_NR * _FOUT,), jnp.float32),
            pltpu.VMEM((_H * _NR,), jnp.float32),
            pltpu.VMEM((_H * 16,), jnp.float32),
            pltpu.SemaphoreType.DMA((_NB,)),
            pltpu.SemaphoreType.DMA((2,)),
        ],
    )(_agg_body)
    return kern(src, dst, wgt, hp, sinv)


def _sinv_body(s_ref, o_ref):
    st = jnp.sum(s_ref[...], axis=0)
    o_ref[...] = jnp.where(st > 0.0, (1.0 / _H) / st, 0.0)


def _sinv(s_all):
    return pl.pallas_call(
        _sinv_body,
        in_specs=[pl.BlockSpec((_NW, _H, _N), lambda: (0, 0, 0))],
        out_specs=pl.BlockSpec((_H, _N), lambda: (0, 0)),
        out_shape=jax.ShapeDtypeStruct((_H, _N), jnp.float32),
    )(s_all)


def _fin_body(un_ref, b_ref, o_ref):
    un = un_ref[...]
    un = un.reshape(_BN, 2, 2, 16).transpose(0, 1, 3, 2)
    un = un.reshape(_BN, _FOUT)
    o_ref[...] = un + b_ref[...][None, :]


def _finalize(un, bias):
    nb = _N // _BN
    return pl.pallas_call(
        _fin_body,
        grid=(nb,),
        in_specs=[
            pl.BlockSpec((_BN, _FOUT), lambda i: (i, 0)),
            pl.BlockSpec((_FOUT,), lambda i: (0,)),
        ],
        out_specs=pl.BlockSpec((_BN, _FOUT), lambda i: (i, 0)),
        out_shape=jax.ShapeDtypeStruct((_N, _FOUT), jnp.float32),
    )(un, bias)


def kernel(h, edge_index, w, fc, bias):
    src = edge_index[0]
    dst = edge_index[1]
    wcat = jnp.transpose(w, (1, 0, 2)).reshape(_FIN, _D)
    fc_src = fc[:, :_FOUT, 0]
    fc_dst = fc[:, _FOUT:, 0]
    fcsd = jnp.zeros((_D, 16), jnp.float32)
    for hh in range(_H):
        fcsd = fcsd.at[hh * _FOUT:(hh + 1) * _FOUT, hh].set(fc_src[hh])
        fcsd = fcsd.at[hh * _FOUT:(hh + 1) * _FOUT, _H + hh].set(fc_dst[hh])

    hp16, aa, md = _project(h, wcat, fcsd)
    hp32 = lax.bitcast_convert_type(hp16.reshape(_N, _D // 2, 2), jnp.int32)
    wgt, s_all = _wgt_phase(src, dst, aa, md)
    sinv = _sinv(s_all.reshape(_NW, _H, _N))
    un = _agg_phase(src, dst, wgt, hp32, sinv)
    return _finalize(un.reshape(_N, _FOUT), bias)
